# baseline (speedup 1.0000x reference)
import functools

import jax
import jax.numpy as jnp
from jax import lax
from jax.experimental import pallas as pl
from jax.experimental.pallas import tpu as pltpu
from jax.experimental.pallas import tpu_sc as plsc

D_MODEL = 64
GROUPS = (104, 96)
ROWS_PER_CHUNK = 2
NBUF = 4
LOOK = 3
SCALE = 8.0
LANES = 16


@functools.lru_cache(maxsize=None)
def _build(batch, hist, vocab):
    info = plsc.get_sparse_core_info()
    nw = info.num_cores * info.num_subcores
    b_per_w = batch // nw
    n_chunks = b_per_w // ROWS_PER_CHUNK

    mesh = plsc.VectorSubcoreMesh(core_axis_name="c", subcore_axis_name="s")

    @functools.partial(
        pl.kernel,
        mesh=mesh,
        out_type=jax.ShapeDtypeStruct((batch, hist, D_MODEL), jnp.float32),
        scratch_types=[
            pltpu.VMEM((b_per_w, hist), jnp.int32),
            pltpu.VMEM((ROWS_PER_CHUNK, hist, D_MODEL), jnp.float32),
            pltpu.VMEM((ROWS_PER_CHUNK, hist, D_MODEL), jnp.float32),
            pltpu.VMEM((ROWS_PER_CHUNK, hist, D_MODEL), jnp.float32),
            pltpu.VMEM((ROWS_PER_CHUNK, hist, D_MODEL), jnp.float32),
            pltpu.SemaphoreType.DMA,
            pltpu.SemaphoreType.DMA,
        ],
        compiler_params=pltpu.CompilerParams(use_tc_tiling_on_sc=False),
    )
    def k(table_hbm, x_hbm, out_hbm, idx_v, b0, b1, b2, b3, gsem, ssem):
        bufs = [b0, b1, b2, b3]
        wid = lax.axis_index("s") * info.num_cores + lax.axis_index("c")
        bbase = wid * b_per_w
        pltpu.sync_copy(x_hbm.at[pl.ds(bbase, b_per_w)], idx_v)

        ghandles = {}
        shandles = {}

        def start_gathers(c):
            p = c % NBUF
            hs = []
            for i in range(ROWS_PER_CHUNK):
                off = 0
                for g in GROUPS:
                    hs.append(pltpu.async_copy(
                        table_hbm.at[idx_v.at[c * ROWS_PER_CHUNK + i,
                                              pl.ds(off, g)]],
                        bufs[p].at[i, pl.ds(off, g)],
                        gsem))
                    off += g
            ghandles[c] = hs

        def scale_chunk(p):
            buf = bufs[p]
            for i in range(ROWS_PER_CHUNK):
                def row_body(r, carry, i=i):
                    for q in range(D_MODEL // LANES):
                        sl = pl.ds(q * LANES, LANES)
                        buf[i, r, sl] = buf[i, r, sl] * SCALE
                    return carry

                lax.fori_loop(0, hist, row_body, 0, unroll=4)

        for c in range(LOOK):
            start_gathers(c)
        for c in range(n_chunks):
            p = c % NBUF
            for h in ghandles.pop(c):
                h.wait()
            scale_chunk(p)
            shandles[c] = pltpu.async_copy(
                bufs[p],
                out_hbm.at[pl.ds(bbase + c * ROWS_PER_CHUNK, ROWS_PER_CHUNK)],
                ssem)
            nxt = c + LOOK
            if nxt < n_chunks:
                prev_user = nxt - NBUF
                if prev_user >= 0:
                    shandles.pop(prev_user).wait()
                start_gathers(nxt)
        for c in sorted(shandles):
            shandles.pop(c).wait()

    return k


def kernel(x, table):
    b, h = x.shape
    return _build(b, h, table.shape[0])(table, x.astype(jnp.int32))
